# Initial kernel scaffold; baseline (speedup 1.0000x reference)
import jax
import jax.numpy as jnp
from jax.experimental import pallas as pl


def kernel(feats, edge_index, edge_attr, Wq, bq, Wk, bk, Wv, bv, Wo, bo, We, be, W1, b1, W2, b2, ln_g, ln_b):
    raise NotImplementedError("write your pallas kernel here")


# speedup vs baseline: 1.0013x; 1.0013x over previous
import math

import jax
import jax.numpy as jnp
from jax.experimental import pallas as pl

N = 10000
E = 320000
EMBED = 128
NHEAD = 8
HDIM = EMBED // NHEAD
DHID = 4 * EMBED


def _layernorm(x, g, b):
    m = jnp.mean(x, axis=-1, keepdims=True)
    v = jnp.mean((x - m) ** 2, axis=-1, keepdims=True)
    return (x - m) / jnp.sqrt(v + 1e-5) * g + b


def _norm(x, g, b):
    return _layernorm(jax.nn.gelu(x, approximate=False), g, b)


def _erf(z):
    s = jnp.sign(z)
    a = jnp.abs(z)
    t = 1.0 / (1.0 + 0.3275911 * a)
    y = 1.0 - (((((1.061405429 * t - 1.453152027) * t) + 1.421413741) * t
                - 0.284496736) * t + 0.254829592) * t * jnp.exp(-a * a)
    return s * y


def _gelu(x):
    return 0.5 * x * (1.0 + _erf(x * 0.7071067811865476))


def _ffn_body(x_ref, w1_ref, b1_ref, w2_ref, b2_ref, o_ref):
    h = _gelu(x_ref[...] @ w1_ref[...] + b1_ref[...])
    o_ref[...] = h @ w2_ref[...] + b2_ref[...]


def _ffn(x, W1, b1, W2, b2):
    blk = 2000
    n = x.shape[0]
    return pl.pallas_call(
        _ffn_body,
        grid=(n // blk,),
        in_specs=[
            pl.BlockSpec((blk, EMBED), lambda i: (i, 0)),
            pl.BlockSpec((EMBED, DHID), lambda i: (0, 0)),
            pl.BlockSpec((1, DHID), lambda i: (0, 0)),
            pl.BlockSpec((DHID, EMBED), lambda i: (0, 0)),
            pl.BlockSpec((1, EMBED), lambda i: (0, 0)),
        ],
        out_specs=pl.BlockSpec((blk, EMBED), lambda i: (i, 0)),
        out_shape=jax.ShapeDtypeStruct((n, EMBED), jnp.float32),
    )(x, W1.T, b1[None], W2.T, b2[None])


def kernel(feats, edge_index, edge_attr, Wq, bq, Wk, bk, Wv, bv, Wo, bo, We, be, W1, b1, W2, b2, ln_g, ln_b):
    bs = feats.shape[0]
    r = edge_index[:, 0]
    c = edge_index[:, 1]
    q = (feats @ Wq.T + bq).reshape(bs, NHEAD, HDIM).transpose(1, 0, 2)
    k = (feats @ Wk.T + bk).reshape(bs, NHEAD, HDIM).transpose(1, 0, 2)
    v = (feats @ Wv.T + bv).reshape(bs, NHEAD, HDIM).transpose(1, 0, 2)
    q_c = jnp.take(q, c, axis=1)
    k_r = jnp.take(k, r, axis=1)
    scores = jnp.sum(q_c * k_r, axis=-1) / math.sqrt(HDIM)
    sc = scores.T
    seg_max = jax.ops.segment_max(sc, c, num_segments=bs)
    ex = jnp.exp(sc - jnp.take(seg_max, c, axis=0))
    seg_sum = jax.ops.segment_sum(ex, c, num_segments=bs)
    attn = ex / (jnp.take(seg_sum, c, axis=0) + 1e-16)
    v_r = jnp.take(v, r, axis=1).transpose(1, 0, 2)
    weighted = attn[:, :, None] * v_r
    agg = jax.ops.segment_sum(weighted.reshape(E, NHEAD * HDIM), c, num_segments=bs)
    att_out = agg @ Wo.T + bo
    feats_ = _norm(feats + att_out, ln_g, ln_b)
    node_attr = jax.nn.gelu(edge_attr @ We.T + be, approximate=False)
    edge_agg = jax.ops.segment_sum(node_attr, r, num_segments=bs)
    feats_att = _norm(feats_ + edge_agg, ln_g, ln_b)
    ffn = _ffn(feats_att, W1, b1, W2, b2)
    out = _norm(feats_att + ffn, ln_g, ln_b)
    return (out, edge_index, edge_attr)


# speedup vs baseline: 4.2031x; 4.1978x over previous
import functools
import math

import jax
import jax.numpy as jnp
import numpy as np
from jax import lax
from jax.experimental import pallas as pl
from jax.experimental.pallas import tpu as pltpu
from jax.experimental.pallas import tpu_sc as plsc

N = 10000
E = 320000
EMBED = 128
NHEAD = 8
HDIM = EMBED // NHEAD
DHID = 4 * EMBED

NC = 2
NS = 16
NW = NC * NS
L = 16

CA = 64
NCHUNK_A = E // CA
NJ_A = (NCHUNK_A + NW - 1) // NW
CE = 128
NCHUNK_E = E // CE
NJ_E = (NCHUNK_E + NW - 1) // NW
ROWS_PER_TILE = 624
REM_ROWS = N - NS * ROWS_PER_TILE

_DNUMS = lax.GatherDimensionNumbers(
    offset_dims=(), collapsed_slice_dims=(0,), start_index_map=(0,))


def _shuf(v, idx):
    return lax.gather(v, idx[:, None], dimension_numbers=_DNUMS,
                      slice_sizes=(1,),
                      mode=lax.GatherScatterMode.PROMISE_IN_BOUNDS)


def _erf(z):
    s = jnp.sign(z)
    a = jnp.abs(z)
    t = 1.0 / (1.0 + 0.3275911 * a)
    y = 1.0 - (((((1.061405429 * t - 1.453152027) * t) + 1.421413741) * t
                - 0.284496736) * t + 0.254829592) * t * jnp.exp(-a * a)
    return s * y


def _gelu(x):
    return 0.5 * x * (1.0 + _erf(x * 0.7071067811865476))


def _ln(x, g, b):
    m = jnp.mean(x, axis=-1, keepdims=True)
    v = jnp.mean((x - m) ** 2, axis=-1, keepdims=True)
    return (x - m) / jnp.sqrt(v + 1e-5) * g + b


def _qkv_body(x_ref, wq_ref, bq_ref, wk_ref, bk_ref, wv_ref, bv_ref,
              q_ref, kv_ref):
    x = x_ref[...]
    q_ref[...] = x @ wq_ref[...] + bq_ref[...]
    kv_ref[:, :EMBED] = (x @ wk_ref[...] + bk_ref[...]) * (1.0 / math.sqrt(HDIM))
    kv_ref[:, EMBED:] = x @ wv_ref[...] + bv_ref[...]


def _qkv(feats, Wq, bq, Wk, bk, Wv, bv):
    blk = 2000
    w_spec = pl.BlockSpec((EMBED, EMBED), lambda i: (0, 0))
    b_spec = pl.BlockSpec((1, EMBED), lambda i: (0, 0))
    return pl.pallas_call(
        _qkv_body,
        grid=(N // blk,),
        in_specs=[pl.BlockSpec((blk, EMBED), lambda i: (i, 0)),
                  w_spec, b_spec, w_spec, b_spec, w_spec, b_spec],
        out_specs=[pl.BlockSpec((blk, EMBED), lambda i: (i, 0)),
                   pl.BlockSpec((blk, 2 * EMBED), lambda i: (i, 0))],
        out_shape=[jax.ShapeDtypeStruct((N, EMBED), jnp.float32),
                   jax.ShapeDtypeStruct((N, 2 * EMBED), jnp.float32)],
    )(feats, Wq.T, bq[None], Wk.T, bk[None], Wv.T, bv[None])


def _na_body(ea_ref, we_ref, be_ref, o_ref):
    o_ref[...] = _gelu(ea_ref[...] @ we_ref[...] + be_ref[...])


def _node_attr(edge_attr, We, be):
    blk = 2560
    return pl.pallas_call(
        _na_body,
        grid=(E // blk,),
        in_specs=[pl.BlockSpec((blk, EMBED), lambda i: (i, 0)),
                  pl.BlockSpec((EMBED, EMBED), lambda i: (0, 0)),
                  pl.BlockSpec((1, EMBED), lambda i: (0, 0))],
        out_specs=pl.BlockSpec((blk, EMBED), lambda i: (i, 0)),
        out_shape=jax.ShapeDtypeStruct((E, EMBED), jnp.float32),
    )(edge_attr, We.T, be[None])


ND = 1256


def _attn_sc_body(q_hbm, kv_hbm, r_hbm, c_hbm,
                  outa_hbm, outd_hbm,
                  qbuf, kvbuf, hswide, cidx, cidx2, ridx, cpad, acca, accd):
    cid = lax.axis_index("c")
    sid = lax.axis_index("s")
    wid = sid * NC + cid

    zv = jnp.zeros((L,), jnp.float32)

    def zrow(i, _):
        for jj in range(EMBED // L):
            qbuf[i, pl.ds(jj * L, L)] = zv
        return 0
    lax.fori_loop(0, CA, zrow, 0)

    base = sid * ROWS_PER_TILE
    zslices = [(off, CA) for off in range(0, 576, CA)] + [(576, 48)]
    for off, nrow in zslices:
        pltpu.sync_copy(qbuf.at[pl.ds(0, nrow)], acca.at[pl.ds(base + off, nrow)])

    @pl.when(sid == 0)
    def _():
        pltpu.sync_copy(qbuf.at[pl.ds(0, REM_ROWS)],
                        acca.at[pl.ds(NS * ROWS_PER_TILE, REM_ROWS)])

    dbase = sid * 80
    dlen = jnp.where(sid == 15, 56, 80)

    @pl.when(sid < 15)
    def _():
        pltpu.sync_copy(qbuf.at[pl.ds(0, 64)], accd.at[pl.ds(dbase, 64)])
        pltpu.sync_copy(qbuf.at[pl.ds(0, 16)], accd.at[pl.ds(dbase + 64, 16)])

    @pl.when(sid == 15)
    def _():
        pltpu.sync_copy(qbuf.at[pl.ds(0, 56)], accd.at[pl.ds(1200, 56)])
    plsc.subcore_barrier()

    lane = lax.iota(jnp.int32, L)
    masks = [jnp.where(lane == h, 1.0, 0.0).astype(jnp.float32)
             for h in range(NHEAD)]
    perms = [lane ^ sh for sh in (8, 4, 2, 1)]

    def chunk(j, _):
        chunk_id = wid + NW * j

        @pl.when(chunk_id < NCHUNK_A)
        def _():
            ebase = chunk_id * CA
            pltpu.sync_copy(c_hbm.at[pl.ds(ebase, CA)], cidx)
            pltpu.sync_copy(c_hbm.at[pl.ds(ebase, CA)], cpad.at[pl.ds(0, CA)])
            pltpu.sync_copy(r_hbm.at[pl.ds(ebase, CA)], ridx)

            for g in range(CA // L):
                cidx2[pl.ds(g * L, L)] = lax.shift_right_logical(
                    cidx[pl.ds(g * L, L)], 3)

            pltpu.sync_copy(q_hbm.at[cidx], qbuf)
            pltpu.sync_copy(kv_hbm.at[ridx], kvbuf)

            def edge(e, _):
                hs = jnp.zeros((L,), jnp.float32)
                for h in range(NHEAD):
                    qv = qbuf[e, pl.ds(h * HDIM, L)]
                    kv = kvbuf[e, pl.ds(h * HDIM, L)]
                    t = qv * kv
                    for p in perms:
                        t = t + _shuf(t, p)
                    ex = jnp.exp(t)
                    vv = kvbuf[e, pl.ds(EMBED + h * HDIM, L)]
                    qbuf[e, pl.ds(h * HDIM, L)] = ex * vv
                    hs = hs + ex * masks[h]
                cv16 = cpad[pl.ds(e, L)]
                slot = jnp.bitwise_and(cv16[0], 7)
                for sslot in range(8):
                    hswide[e, pl.ds(sslot * L, L)] = zv
                hswide[e, pl.ds(slot * L, L)] = hs
                return 0
            lax.fori_loop(0, CA, edge, 0)

            pltpu.sync_copy(qbuf, acca.at[cidx], add=True)
            pltpu.sync_copy(hswide, accd.at[cidx2], add=True)
        return 0

    lax.fori_loop(0, NJ_A, chunk, 0)

    plsc.subcore_barrier()
    pltpu.sync_copy(acca.at[pl.ds(base, ROWS_PER_TILE)],
                    outa_hbm.at[cid, pl.ds(base, ROWS_PER_TILE)])

    @pl.when(sid == 0)
    def _():
        tail = NS * ROWS_PER_TILE
        pltpu.sync_copy(acca.at[pl.ds(tail, REM_ROWS)],
                        outa_hbm.at[cid, pl.ds(tail, REM_ROWS)])

    @pl.when(sid < 15)
    def _():
        pltpu.sync_copy(accd.at[pl.ds(dbase, 80)],
                        outd_hbm.at[cid, pl.ds(dbase, 80)])

    @pl.when(sid == 15)
    def _():
        pltpu.sync_copy(accd.at[pl.ds(1200, 56)],
                        outd_hbm.at[cid, pl.ds(1200, 56)])


def _attn_sc(q, kv, r, c):
    mesh = plsc.VectorSubcoreMesh(core_axis_name="c", subcore_axis_name="s")
    return pl.kernel(
        _attn_sc_body,
        out_type=[jax.ShapeDtypeStruct((NC, N, EMBED), jnp.float32),
                  jax.ShapeDtypeStruct((NC, ND, EMBED), jnp.float32)],
        mesh=mesh,
        scratch_types=[
            pltpu.VMEM((CA, EMBED), jnp.float32),
            pltpu.VMEM((CA, 2 * EMBED), jnp.float32),
            pltpu.VMEM((CA, EMBED), jnp.float32),
            pltpu.VMEM((CA,), jnp.int32),
            pltpu.VMEM((CA,), jnp.int32),
            pltpu.VMEM((CA,), jnp.int32),
            pltpu.VMEM((CA + L,), jnp.int32),
            pltpu.VMEM_SHARED((N, EMBED), jnp.float32),
            pltpu.VMEM_SHARED((ND, EMBED), jnp.float32),
        ],
    )(q, kv, r, c)


def _eagg_sc_body(na_hbm, r_hbm, out_hbm, buf, ridx, acc):
    cid = lax.axis_index("c")
    sid = lax.axis_index("s")
    wid = sid * NC + cid

    zv = jnp.zeros((L,), jnp.float32)

    def zrow(i, _):
        for jj in range(EMBED // L):
            buf[i, pl.ds(jj * L, L)] = zv
        return 0
    lax.fori_loop(0, CE, zrow, 0)

    base = sid * ROWS_PER_TILE
    for off, nrow in ((0, 128), (128, 128), (256, 128), (384, 128), (512, 112)):
        pltpu.sync_copy(buf.at[pl.ds(0, nrow)], acc.at[pl.ds(base + off, nrow)])

    @pl.when(sid == 0)
    def _():
        pltpu.sync_copy(buf.at[pl.ds(0, REM_ROWS)],
                        acc.at[pl.ds(NS * ROWS_PER_TILE, REM_ROWS)])
    plsc.subcore_barrier()

    def chunk(j, _):
        chunk_id = wid + NW * j

        @pl.when(chunk_id < NCHUNK_E)
        def _():
            ebase = chunk_id * CE
            pltpu.sync_copy(r_hbm.at[pl.ds(ebase, CE)], ridx)
            pltpu.sync_copy(na_hbm.at[pl.ds(ebase, CE)], buf)
            pltpu.sync_copy(buf, acc.at[ridx], add=True)
        return 0

    lax.fori_loop(0, NJ_E, chunk, 0)

    plsc.subcore_barrier()
    pltpu.sync_copy(acc.at[pl.ds(base, ROWS_PER_TILE)],
                    out_hbm.at[cid, pl.ds(base, ROWS_PER_TILE)])

    @pl.when(sid == 0)
    def _():
        tail = NS * ROWS_PER_TILE
        pltpu.sync_copy(acc.at[pl.ds(tail, REM_ROWS)],
                        out_hbm.at[cid, pl.ds(tail, REM_ROWS)])


def _eagg_sc(node_attr, r):
    mesh = plsc.VectorSubcoreMesh(core_axis_name="c", subcore_axis_name="s")
    return pl.kernel(
        _eagg_sc_body,
        out_type=jax.ShapeDtypeStruct((NC, N, EMBED), jnp.float32),
        mesh=mesh,
        scratch_types=[
            pltpu.VMEM((CE, EMBED), jnp.float32),
            pltpu.VMEM((CE,), jnp.int32),
            pltpu.VMEM_SHARED((N, EMBED), jnp.float32),
        ],
    )(node_attr, r)


def _fin_body(feats_ref, acca_ref, accb_ref, eagg_ref, m8_ref,
              wo_ref, bo_ref, w1_ref, b1_ref, w2_ref, b2_ref,
              g_ref, b_ref, o_ref):
    g = g_ref[...]
    b = b_ref[...]
    num = acca_ref[0] + acca_ref[1]
    den = accb_ref[0] + accb_ref[1]
    rec = 1.0 / (den + 1e-16)
    r128 = jax.lax.dot(rec, m8_ref[...], preferred_element_type=jnp.float32)
    agg = num * r128
    att = agg @ wo_ref[...] + bo_ref[...]
    f_ = _ln(_gelu(feats_ref[...] + att), g, b)
    fa = _ln(_gelu(f_ + eagg_ref[0] + eagg_ref[1]), g, b)
    ffn = _gelu(fa @ w1_ref[...] + b1_ref[...]) @ w2_ref[...] + b2_ref[...]
    o_ref[...] = _ln(_gelu(fa + ffn), g, b)


def _finalize(feats, acca, accb, eagg, Wo, bo, W1, b1, W2, b2, ln_g, ln_b):
    blk = 2000
    m8 = jnp.asarray(np.repeat(np.eye(NHEAD, dtype=np.float32), HDIM, axis=1))
    w_spec = lambda shape: pl.BlockSpec(shape, lambda i: (0,) * len(shape))
    return pl.pallas_call(
        _fin_body,
        grid=(N // blk,),
        in_specs=[
            pl.BlockSpec((blk, EMBED), lambda i: (i, 0)),
            pl.BlockSpec((NC, blk, EMBED), lambda i: (0, i, 0)),
            pl.BlockSpec((NC, blk, NHEAD), lambda i: (0, i, 0)),
            pl.BlockSpec((NC, blk, EMBED), lambda i: (0, i, 0)),
            w_spec((NHEAD, EMBED)),
            w_spec((EMBED, EMBED)), w_spec((1, EMBED)),
            w_spec((EMBED, DHID)), w_spec((1, DHID)),
            w_spec((DHID, EMBED)), w_spec((1, EMBED)),
            w_spec((1, EMBED)), w_spec((1, EMBED)),
        ],
        out_specs=pl.BlockSpec((blk, EMBED), lambda i: (i, 0)),
        out_shape=jax.ShapeDtypeStruct((N, EMBED), jnp.float32),
    )(feats, acca, accb, eagg, m8, Wo.T, bo[None], W1.T, b1[None],
      W2.T, b2[None], ln_g[None], ln_b[None])


def kernel(feats, edge_index, edge_attr, Wq, bq, Wk, bk, Wv, bv, Wo, bo,
           We, be, W1, b1, W2, b2, ln_g, ln_b):
    r = edge_index[:, 0]
    c = edge_index[:, 1]
    q, kv = _qkv(feats, Wq, bq, Wk, bk, Wv, bv)
    acca, outd = _attn_sc(q, kv, r, c)
    den = outd.reshape(NC, ND, 8, 16)[:, :, :, :NHEAD].reshape(NC, ND * 8, NHEAD)[:, :N, :]
    na = _node_attr(edge_attr, We, be)
    eagg = _eagg_sc(na, r)
    out = _finalize(feats, acca, den, eagg, Wo, bo, W1, b1, W2, b2,
                    ln_g, ln_b)
    return (out, edge_index, edge_attr)


# speedup vs baseline: 4.8620x; 1.1568x over previous
import functools
import math

import jax
import jax.numpy as jnp
import numpy as np
from jax import lax
from jax.experimental import pallas as pl
from jax.experimental.pallas import tpu as pltpu
from jax.experimental.pallas import tpu_sc as plsc

N = 10000
E = 320000
EMBED = 128
NHEAD = 8
HDIM = EMBED // NHEAD
DHID = 4 * EMBED

NC = 2
NS = 16
NW = NC * NS
L = 16

CA = 64
NCHUNK_A = E // CA
NJ_A = (NCHUNK_A + NW - 1) // NW
CE = 128
NCHUNK_E = E // CE
NJ_E = (NCHUNK_E + NW - 1) // NW
ROWS_PER_TILE = 624
REM_ROWS = N - NS * ROWS_PER_TILE

_DNUMS = lax.GatherDimensionNumbers(
    offset_dims=(), collapsed_slice_dims=(0,), start_index_map=(0,))


def _shuf(v, idx):
    return lax.gather(v, idx[:, None], dimension_numbers=_DNUMS,
                      slice_sizes=(1,),
                      mode=lax.GatherScatterMode.PROMISE_IN_BOUNDS)


def _erf(z):
    s = jnp.sign(z)
    a = jnp.abs(z)
    t = 1.0 / (1.0 + 0.3275911 * a)
    y = 1.0 - (((((1.061405429 * t - 1.453152027) * t) + 1.421413741) * t
                - 0.284496736) * t + 0.254829592) * t * jnp.exp(-a * a)
    return s * y


def _gelu(x):
    return 0.5 * x * (1.0 + _erf(x * 0.7071067811865476))


def _ln(x, g, b):
    m = jnp.mean(x, axis=-1, keepdims=True)
    v = jnp.mean((x - m) ** 2, axis=-1, keepdims=True)
    return (x - m) / jnp.sqrt(v + 1e-5) * g + b


def _qkv_body(x_ref, wq_ref, bq_ref, wk_ref, bk_ref, wv_ref, bv_ref,
              q_ref, kv_ref):
    x = x_ref[...]
    q_ref[...] = x @ wq_ref[...] + bq_ref[...]
    kv_ref[:, :EMBED] = (x @ wk_ref[...] + bk_ref[...]) * (1.0 / math.sqrt(HDIM))
    kv_ref[:, EMBED:] = x @ wv_ref[...] + bv_ref[...]


def _qkv(feats, Wq, bq, Wk, bk, Wv, bv):
    blk = 2000
    w_spec = pl.BlockSpec((EMBED, EMBED), lambda i: (0, 0))
    b_spec = pl.BlockSpec((1, EMBED), lambda i: (0, 0))
    return pl.pallas_call(
        _qkv_body,
        grid=(N // blk,),
        in_specs=[pl.BlockSpec((blk, EMBED), lambda i: (i, 0)),
                  w_spec, b_spec, w_spec, b_spec, w_spec, b_spec],
        out_specs=[pl.BlockSpec((blk, EMBED), lambda i: (i, 0)),
                   pl.BlockSpec((blk, 2 * EMBED), lambda i: (i, 0))],
        out_shape=[jax.ShapeDtypeStruct((N, EMBED), jnp.float32),
                   jax.ShapeDtypeStruct((N, 2 * EMBED), jnp.float32)],
    )(feats, Wq.T, bq[None], Wk.T, bk[None], Wv.T, bv[None])


def _na_body(ea_ref, we_ref, be_ref, o_ref):
    o_ref[...] = _gelu(ea_ref[...] @ we_ref[...] + be_ref[...])


def _node_attr(edge_attr, We, be):
    blk = 2560
    return pl.pallas_call(
        _na_body,
        grid=(E // blk,),
        in_specs=[pl.BlockSpec((blk, EMBED), lambda i: (i, 0)),
                  pl.BlockSpec((EMBED, EMBED), lambda i: (0, 0)),
                  pl.BlockSpec((1, EMBED), lambda i: (0, 0))],
        out_specs=pl.BlockSpec((blk, EMBED), lambda i: (i, 0)),
        out_shape=jax.ShapeDtypeStruct((E, EMBED), jnp.float32),
    )(edge_attr, We.T, be[None])


ND = 1256


def _attn_sc_body(q_hbm, kv_hbm, r_hbm, c_hbm,
                  outa_hbm, outd_hbm,
                  qbuf, kvbuf, hswide, cidx, cidx2, ridx, cpad,
                  isem, gsem, ssem, acca, accd):
    cid = lax.axis_index("c")
    sid = lax.axis_index("s")
    wid = sid * NC + cid

    zv = jnp.zeros((L,), jnp.float32)

    def zrow(i, _):
        for jj in range(EMBED // L):
            qbuf[i, pl.ds(jj * L, L)] = zv
        return 0
    lax.fori_loop(0, CA, zrow, 0)

    base = sid * ROWS_PER_TILE
    zslices = [(off, CA) for off in range(0, 576, CA)] + [(576, 48)]
    for off, nrow in zslices:
        pltpu.sync_copy(qbuf.at[pl.ds(0, nrow)], acca.at[pl.ds(base + off, nrow)])

    @pl.when(sid == 0)
    def _():
        pltpu.sync_copy(qbuf.at[pl.ds(0, REM_ROWS)],
                        acca.at[pl.ds(NS * ROWS_PER_TILE, REM_ROWS)])

    dbase = sid * 80

    @pl.when(sid < 15)
    def _():
        pltpu.sync_copy(qbuf.at[pl.ds(0, 64)], accd.at[pl.ds(dbase, 64)])
        pltpu.sync_copy(qbuf.at[pl.ds(0, 16)], accd.at[pl.ds(dbase + 64, 16)])

    @pl.when(sid == 15)
    def _():
        pltpu.sync_copy(qbuf.at[pl.ds(0, 56)], accd.at[pl.ds(1200, 56)])
    plsc.subcore_barrier()

    lane = lax.iota(jnp.int32, L)
    masks = [jnp.where(lane == h, 1.0, 0.0).astype(jnp.float32)
             for h in range(NHEAD)]
    perms = [lane ^ sh for sh in (8, 4, 2, 1)]

    def issue_idx(jn):
        sn = lax.rem(jn, 2)
        ebase = (wid + NW * jn) * CA
        pltpu.async_copy(c_hbm.at[pl.ds(ebase, CA)], cidx.at[sn], isem)
        pltpu.async_copy(c_hbm.at[pl.ds(ebase, CA)], cpad.at[sn, pl.ds(0, CA)], isem)
        pltpu.async_copy(r_hbm.at[pl.ds(ebase, CA)], ridx.at[sn], isem)

    issue_idx(0)

    def chunk(j, _):
        sl = lax.rem(j, 2)
        sp = 1 - sl
        valid = (wid + NW * j) < NCHUNK_A
        valid_n = (wid + NW * (j + 1)) < NCHUNK_A

        @pl.when(j >= 1)
        def _():
            pltpu.make_async_copy(qbuf, acca.at[cidx.at[sp]], ssem).wait()
            pltpu.make_async_copy(hswide, accd.at[cidx2.at[sp]], ssem).wait()

        @pl.when(valid_n)
        def _():
            issue_idx(j + 1)

        @pl.when(valid)
        def _():
            pltpu.make_async_copy(c_hbm.at[pl.ds(0, CA)], cidx.at[sl], isem).wait()
            pltpu.make_async_copy(c_hbm.at[pl.ds(0, CA)], cpad.at[sl, pl.ds(0, CA)], isem).wait()
            pltpu.make_async_copy(r_hbm.at[pl.ds(0, CA)], ridx.at[sl], isem).wait()

            for g in range(CA // L):
                cidx2[sl, pl.ds(g * L, L)] = lax.shift_right_logical(
                    cidx[sl, pl.ds(g * L, L)], 3)

            dq = pltpu.async_copy(q_hbm.at[cidx.at[sl]], qbuf, gsem)
            dkv = pltpu.async_copy(kv_hbm.at[ridx.at[sl]], kvbuf, gsem)
            dq.wait()
            dkv.wait()

            def edge(e, _):
                hs = jnp.zeros((L,), jnp.float32)
                for h in range(NHEAD):
                    qv = qbuf[e, pl.ds(h * HDIM, L)]
                    kv = kvbuf[e, pl.ds(h * HDIM, L)]
                    t = qv * kv
                    for pp in perms:
                        t = t + _shuf(t, pp)
                    ex = jnp.exp(t)
                    vv = kvbuf[e, pl.ds(EMBED + h * HDIM, L)]
                    qbuf[e, pl.ds(h * HDIM, L)] = ex * vv
                    hs = hs + ex * masks[h]
                cv16 = cpad[sl, pl.ds(e, L)]
                slot = jnp.bitwise_and(cv16[0], 7)
                for sslot in range(8):
                    hswide[e, pl.ds(sslot * L, L)] = zv
                hswide[e, pl.ds(slot * L, L)] = hs
                return 0
            lax.fori_loop(0, CA, edge, 0)

            pltpu.async_copy(qbuf, acca.at[cidx.at[sl]], ssem, add=True)
            pltpu.async_copy(hswide, accd.at[cidx2.at[sl]], ssem, add=True)
        return 0

    lax.fori_loop(0, NJ_A, chunk, 0)

    lastsl = (NJ_A - 1) % 2

    @pl.when(wid < NCHUNK_A - NW * (NJ_A - 1))
    def _():
        pltpu.make_async_copy(qbuf, acca.at[cidx.at[lastsl]], ssem).wait()
        pltpu.make_async_copy(hswide, accd.at[cidx2.at[lastsl]], ssem).wait()

    plsc.subcore_barrier()
    pltpu.sync_copy(acca.at[pl.ds(base, ROWS_PER_TILE)],
                    outa_hbm.at[cid, pl.ds(base, ROWS_PER_TILE)])

    @pl.when(sid == 0)
    def _():
        tail = NS * ROWS_PER_TILE
        pltpu.sync_copy(acca.at[pl.ds(tail, REM_ROWS)],
                        outa_hbm.at[cid, pl.ds(tail, REM_ROWS)])

    @pl.when(sid < 15)
    def _():
        pltpu.sync_copy(accd.at[pl.ds(dbase, 80)],
                        outd_hbm.at[cid, pl.ds(dbase, 80)])

    @pl.when(sid == 15)
    def _():
        pltpu.sync_copy(accd.at[pl.ds(1200, 56)],
                        outd_hbm.at[cid, pl.ds(1200, 56)])


def _attn_sc(q, kv, r, c):
    mesh = plsc.VectorSubcoreMesh(core_axis_name="c", subcore_axis_name="s")
    return pl.kernel(
        _attn_sc_body,
        out_type=[jax.ShapeDtypeStruct((NC, N, EMBED), jnp.float32),
                  jax.ShapeDtypeStruct((NC, ND, EMBED), jnp.float32)],
        mesh=mesh,
        scratch_types=[
            pltpu.VMEM((CA, EMBED), jnp.float32),
            pltpu.VMEM((CA, 2 * EMBED), jnp.float32),
            pltpu.VMEM((CA, EMBED), jnp.float32),
            pltpu.VMEM((2, CA), jnp.int32),
            pltpu.VMEM((2, CA), jnp.int32),
            pltpu.VMEM((2, CA), jnp.int32),
            pltpu.VMEM((2, CA + L), jnp.int32),
            pltpu.SemaphoreType.DMA,
            pltpu.SemaphoreType.DMA,
            pltpu.SemaphoreType.DMA,
            pltpu.VMEM_SHARED((N, EMBED), jnp.float32),
            pltpu.VMEM_SHARED((ND, EMBED), jnp.float32),
        ],
    )(q, kv, r, c)


def _eagg_sc_body(na_hbm, r_hbm, out_hbm, buf, ridx, isem, ssem, acc):
    cid = lax.axis_index("c")
    sid = lax.axis_index("s")
    wid = sid * NC + cid

    zv = jnp.zeros((L,), jnp.float32)

    def zrow(i, _):
        for jj in range(EMBED // L):
            buf[0, i, pl.ds(jj * L, L)] = zv
        return 0
    lax.fori_loop(0, CE, zrow, 0)

    base = sid * ROWS_PER_TILE
    for off, nrow in ((0, 128), (128, 128), (256, 128), (384, 128), (512, 112)):
        pltpu.sync_copy(buf.at[0, pl.ds(0, nrow)], acc.at[pl.ds(base + off, nrow)])

    @pl.when(sid == 0)
    def _():
        pltpu.sync_copy(buf.at[0, pl.ds(0, REM_ROWS)],
                        acc.at[pl.ds(NS * ROWS_PER_TILE, REM_ROWS)])
    plsc.subcore_barrier()

    def issue_load(jn):
        sn = lax.rem(jn, 2)
        ebase = (wid + NW * jn) * CE
        pltpu.async_copy(r_hbm.at[pl.ds(ebase, CE)], ridx.at[sn], isem)
        pltpu.async_copy(na_hbm.at[pl.ds(ebase, CE)], buf.at[sn], isem)

    issue_load(0)

    def chunk(j, _):
        sl = lax.rem(j, 2)
        sp = 1 - sl
        valid = (wid + NW * j) < NCHUNK_E
        valid_n = (wid + NW * (j + 1)) < NCHUNK_E

        @pl.when(j >= 1)
        def _():
            pltpu.make_async_copy(buf.at[sp], acc.at[ridx.at[sp]], ssem).wait()

        @pl.when(valid_n)
        def _():
            issue_load(j + 1)

        @pl.when(valid)
        def _():
            pltpu.make_async_copy(r_hbm.at[pl.ds(0, CE)], ridx.at[sl], isem).wait()
            pltpu.make_async_copy(na_hbm.at[pl.ds(0, CE)], buf.at[sl], isem).wait()
            pltpu.async_copy(buf.at[sl], acc.at[ridx.at[sl]], ssem, add=True)
        return 0

    lax.fori_loop(0, NJ_E, chunk, 0)

    lastsl = (NJ_E - 1) % 2

    @pl.when(wid < NCHUNK_E - NW * (NJ_E - 1))
    def _():
        pltpu.make_async_copy(buf.at[lastsl], acc.at[ridx.at[lastsl]], ssem).wait()

    plsc.subcore_barrier()
    pltpu.sync_copy(acc.at[pl.ds(base, ROWS_PER_TILE)],
                    out_hbm.at[cid, pl.ds(base, ROWS_PER_TILE)])

    @pl.when(sid == 0)
    def _():
        tail = NS * ROWS_PER_TILE
        pltpu.sync_copy(acc.at[pl.ds(tail, REM_ROWS)],
                        out_hbm.at[cid, pl.ds(tail, REM_ROWS)])


def _eagg_sc(node_attr, r):
    mesh = plsc.VectorSubcoreMesh(core_axis_name="c", subcore_axis_name="s")
    return pl.kernel(
        _eagg_sc_body,
        out_type=jax.ShapeDtypeStruct((NC, N, EMBED), jnp.float32),
        mesh=mesh,
        scratch_types=[
            pltpu.VMEM((2, CE, EMBED), jnp.float32),
            pltpu.VMEM((2, CE), jnp.int32),
            pltpu.SemaphoreType.DMA,
            pltpu.SemaphoreType.DMA,
            pltpu.VMEM_SHARED((N, EMBED), jnp.float32),
        ],
    )(node_attr, r)


def _fin_body(feats_ref, acca_ref, accb_ref, eagg_ref, m8_ref,
              wo_ref, bo_ref, w1_ref, b1_ref, w2_ref, b2_ref,
              g_ref, b_ref, o_ref):
    g = g_ref[...]
    b = b_ref[...]
    num = acca_ref[0] + acca_ref[1]
    den = accb_ref[0] + accb_ref[1]
    rec = 1.0 / (den + 1e-16)
    r128 = jax.lax.dot(rec, m8_ref[...], preferred_element_type=jnp.float32)
    agg = num * r128
    att = agg @ wo_ref[...] + bo_ref[...]
    f_ = _ln(_gelu(feats_ref[...] + att), g, b)
    fa = _ln(_gelu(f_ + eagg_ref[0] + eagg_ref[1]), g, b)
    ffn = _gelu(fa @ w1_ref[...] + b1_ref[...]) @ w2_ref[...] + b2_ref[...]
    o_ref[...] = _ln(_gelu(fa + ffn), g, b)


def _finalize(feats, acca, accb, eagg, Wo, bo, W1, b1, W2, b2, ln_g, ln_b):
    blk = 2000
    m8 = jnp.asarray(np.repeat(np.eye(NHEAD, dtype=np.float32), HDIM, axis=1))
    w_spec = lambda shape: pl.BlockSpec(shape, lambda i: (0,) * len(shape))
    return pl.pallas_call(
        _fin_body,
        grid=(N // blk,),
        in_specs=[
            pl.BlockSpec((blk, EMBED), lambda i: (i, 0)),
            pl.BlockSpec((NC, blk, EMBED), lambda i: (0, i, 0)),
            pl.BlockSpec((NC, blk, NHEAD), lambda i: (0, i, 0)),
            pl.BlockSpec((NC, blk, EMBED), lambda i: (0, i, 0)),
            w_spec((NHEAD, EMBED)),
            w_spec((EMBED, EMBED)), w_spec((1, EMBED)),
            w_spec((EMBED, DHID)), w_spec((1, DHID)),
            w_spec((DHID, EMBED)), w_spec((1, EMBED)),
            w_spec((1, EMBED)), w_spec((1, EMBED)),
        ],
        out_specs=pl.BlockSpec((blk, EMBED), lambda i: (i, 0)),
        out_shape=jax.ShapeDtypeStruct((N, EMBED), jnp.float32),
    )(feats, acca, accb, eagg, m8, Wo.T, bo[None], W1.T, b1[None],
      W2.T, b2[None], ln_g[None], ln_b[None])


def kernel(feats, edge_index, edge_attr, Wq, bq, Wk, bk, Wv, bv, Wo, bo,
           We, be, W1, b1, W2, b2, ln_g, ln_b):
    r = edge_index[:, 0]
    c = edge_index[:, 1]
    q, kv = _qkv(feats, Wq, bq, Wk, bk, Wv, bv)
    acca, outd = _attn_sc(q, kv, r, c)
    den = outd.reshape(NC, ND, 8, 16)[:, :, :, :NHEAD].reshape(NC, ND * 8, NHEAD)[:, :N, :]
    na = _node_attr(edge_attr, We, be)
    eagg = _eagg_sc(na, r)
    out = _finalize(feats, acca, den, eagg, Wo, bo, W1, b1, W2, b2,
                    ln_g, ln_b)
    return (out, edge_index, edge_attr)


# speedup vs baseline: 12.5176x; 2.5746x over previous
import functools
import math

import jax
import jax.numpy as jnp
import numpy as np
from jax import lax
from jax.experimental import pallas as pl
from jax.experimental.pallas import tpu as pltpu
from jax.experimental.pallas import tpu_sc as plsc

N = 10000
E = 320000
EMBED = 128
NHEAD = 8
HDIM = EMBED // NHEAD
DHID = 4 * EMBED

NC = 2
NS = 16
NW = NC * NS
L = 16

CA = 64
NCHUNK_A = E // CA
NJ_A = (NCHUNK_A + NW - 1) // NW
CE = 128
NCHUNK_E = E // CE
NJ_E = (NCHUNK_E + NW - 1) // NW
ROWS_PER_TILE = 624
REM_ROWS = N - NS * ROWS_PER_TILE

_DNUMS = lax.GatherDimensionNumbers(
    offset_dims=(), collapsed_slice_dims=(0,), start_index_map=(0,))


def _shuf(v, idx):
    return lax.gather(v, idx[:, None], dimension_numbers=_DNUMS,
                      slice_sizes=(1,),
                      mode=lax.GatherScatterMode.PROMISE_IN_BOUNDS)


def _erf(z):
    s = jnp.sign(z)
    a = jnp.abs(z)
    t = 1.0 / (1.0 + 0.3275911 * a)
    y = 1.0 - (((((1.061405429 * t - 1.453152027) * t) + 1.421413741) * t
                - 0.284496736) * t + 0.254829592) * t * jnp.exp(-a * a)
    return s * y


def _gelu(x):
    return 0.5 * x * (1.0 + _erf(x * 0.7071067811865476))


def _ln(x, g, b):
    m = jnp.mean(x, axis=-1, keepdims=True)
    v = jnp.mean((x - m) ** 2, axis=-1, keepdims=True)
    return (x - m) / jnp.sqrt(v + 1e-5) * g + b


def _qkv_body(x_ref, wq_ref, bq_ref, wk_ref, bk_ref, wv_ref, bv_ref,
              q_ref, kv_ref):
    x = x_ref[...]
    q_ref[...] = x @ wq_ref[...] + bq_ref[...]
    kv_ref[:, :EMBED] = (x @ wk_ref[...] + bk_ref[...]) * (1.0 / math.sqrt(HDIM))
    kv_ref[:, EMBED:] = x @ wv_ref[...] + bv_ref[...]


def _qkv(feats, Wq, bq, Wk, bk, Wv, bv):
    blk = 2000
    w_spec = pl.BlockSpec((EMBED, EMBED), lambda i: (0, 0))
    b_spec = pl.BlockSpec((1, EMBED), lambda i: (0, 0))
    return pl.pallas_call(
        _qkv_body,
        grid=(N // blk,),
        in_specs=[pl.BlockSpec((blk, EMBED), lambda i: (i, 0)),
                  w_spec, b_spec, w_spec, b_spec, w_spec, b_spec],
        out_specs=[pl.BlockSpec((blk, EMBED), lambda i: (i, 0)),
                   pl.BlockSpec((blk, 2 * EMBED), lambda i: (i, 0))],
        out_shape=[jax.ShapeDtypeStruct((N, EMBED), jnp.float32),
                   jax.ShapeDtypeStruct((N, 2 * EMBED), jnp.float32)],
    )(feats, Wq.T, bq[None], Wk.T, bk[None], Wv.T, bv[None])


def _na_body(ea_ref, we_ref, be_ref, o_ref):
    o_ref[...] = _gelu(ea_ref[...] @ we_ref[...] + be_ref[...])


def _node_attr(edge_attr, We, be):
    blk = 2560
    return pl.pallas_call(
        _na_body,
        grid=(E // blk,),
        in_specs=[pl.BlockSpec((blk, EMBED), lambda i: (i, 0)),
                  pl.BlockSpec((EMBED, EMBED), lambda i: (0, 0)),
                  pl.BlockSpec((1, EMBED), lambda i: (0, 0))],
        out_specs=pl.BlockSpec((blk, EMBED), lambda i: (i, 0)),
        out_shape=jax.ShapeDtypeStruct((E, EMBED), jnp.float32),
    )(edge_attr, We.T, be[None])


ND = 1256


def _attn_sc_body(q_hbm, kv_hbm, r_hbm, c_hbm,
                  outa_hbm, outd_hbm,
                  qbuf, kvbuf, hswide, cidx, cidx2, ridx, cpad,
                  isem, gsem, ssem, acca, accd):
    cid = lax.axis_index("c")
    sid = lax.axis_index("s")
    wid = sid * NC + cid

    zv = jnp.zeros((L,), jnp.float32)

    def zrow(i, _):
        for jj in range(EMBED // L):
            qbuf[i, pl.ds(jj * L, L)] = zv
        return 0
    lax.fori_loop(0, CA, zrow, 0)

    base = sid * ROWS_PER_TILE
    zslices = [(off, CA) for off in range(0, 576, CA)] + [(576, 48)]
    for off, nrow in zslices:
        pltpu.sync_copy(qbuf.at[pl.ds(0, nrow)], acca.at[pl.ds(base + off, nrow)])

    @pl.when(sid == 0)
    def _():
        pltpu.sync_copy(qbuf.at[pl.ds(0, REM_ROWS)],
                        acca.at[pl.ds(NS * ROWS_PER_TILE, REM_ROWS)])

    dbase = sid * 80

    @pl.when(sid < 15)
    def _():
        pltpu.sync_copy(qbuf.at[pl.ds(0, 64)], accd.at[pl.ds(dbase, 64)])
        pltpu.sync_copy(qbuf.at[pl.ds(0, 16)], accd.at[pl.ds(dbase + 64, 16)])

    @pl.when(sid == 15)
    def _():
        pltpu.sync_copy(qbuf.at[pl.ds(0, 56)], accd.at[pl.ds(1200, 56)])
    plsc.subcore_barrier()

    lane = lax.iota(jnp.int32, L)
    masks = [jnp.where(lane == h, 1.0, 0.0).astype(jnp.float32)
             for h in range(NHEAD)]
    perms = [lane ^ sh for sh in (8, 4, 2, 1)]

    def issue_idx(jn):
        sn = lax.rem(jn, 2)
        ebase = (wid + NW * jn) * CA
        pltpu.async_copy(c_hbm.at[pl.ds(ebase, CA)], cidx.at[sn], isem)
        pltpu.async_copy(c_hbm.at[pl.ds(ebase, CA)], cpad.at[sn, pl.ds(0, CA)], isem)
        pltpu.async_copy(r_hbm.at[pl.ds(ebase, CA)], ridx.at[sn], isem)

    issue_idx(0)

    def chunk(j, _):
        sl = lax.rem(j, 2)
        sp = 1 - sl
        valid = (wid + NW * j) < NCHUNK_A
        valid_n = (wid + NW * (j + 1)) < NCHUNK_A

        @pl.when(j >= 1)
        def _():
            pltpu.make_async_copy(qbuf, acca.at[cidx.at[sp]], ssem).wait()
            pltpu.make_async_copy(hswide, accd.at[cidx2.at[sp]], ssem).wait()

        @pl.when(valid_n)
        def _():
            issue_idx(j + 1)

        @pl.when(valid)
        def _():
            pltpu.make_async_copy(c_hbm.at[pl.ds(0, CA)], cidx.at[sl], isem).wait()
            pltpu.make_async_copy(c_hbm.at[pl.ds(0, CA)], cpad.at[sl, pl.ds(0, CA)], isem).wait()
            pltpu.make_async_copy(r_hbm.at[pl.ds(0, CA)], ridx.at[sl], isem).wait()

            for g in range(CA // L):
                cidx2[sl, pl.ds(g * L, L)] = lax.shift_right_logical(
                    cidx[sl, pl.ds(g * L, L)], 3)

            dq = pltpu.async_copy(q_hbm.at[cidx.at[sl]], qbuf, gsem)
            dkv = pltpu.async_copy(kv_hbm.at[ridx.at[sl]], kvbuf, gsem)
            dq.wait()
            dkv.wait()

            @plsc.parallel_loop(0, CA, unroll=4)
            def _(e):
                hs = jnp.zeros((L,), jnp.float32)
                for h in range(NHEAD):
                    qv = qbuf[e, pl.ds(h * HDIM, L)]
                    kv = kvbuf[e, pl.ds(h * HDIM, L)]
                    t = qv * kv
                    for pp in perms:
                        t = t + _shuf(t, pp)
                    ex = jnp.exp(t)
                    vv = kvbuf[e, pl.ds(EMBED + h * HDIM, L)]
                    qbuf[e, pl.ds(h * HDIM, L)] = ex * vv
                    hs = hs + ex * masks[h]
                cv16 = cpad[sl, pl.ds(e, L)]
                slot = jnp.bitwise_and(cv16[0], 7)
                for sslot in range(8):
                    hswide[e, pl.ds(sslot * L, L)] = zv
                hswide[e, pl.ds(slot * L, L)] = hs

            pltpu.async_copy(qbuf, acca.at[cidx.at[sl]], ssem, add=True)
            pltpu.async_copy(hswide, accd.at[cidx2.at[sl]], ssem, add=True)
        return 0

    lax.fori_loop(0, NJ_A, chunk, 0)

    lastsl = (NJ_A - 1) % 2

    @pl.when(wid < NCHUNK_A - NW * (NJ_A - 1))
    def _():
        pltpu.make_async_copy(qbuf, acca.at[cidx.at[lastsl]], ssem).wait()
        pltpu.make_async_copy(hswide, accd.at[cidx2.at[lastsl]], ssem).wait()

    plsc.subcore_barrier()
    pltpu.sync_copy(acca.at[pl.ds(base, ROWS_PER_TILE)],
                    outa_hbm.at[cid, pl.ds(base, ROWS_PER_TILE)])

    @pl.when(sid == 0)
    def _():
        tail = NS * ROWS_PER_TILE
        pltpu.sync_copy(acca.at[pl.ds(tail, REM_ROWS)],
                        outa_hbm.at[cid, pl.ds(tail, REM_ROWS)])

    @pl.when(sid < 15)
    def _():
        pltpu.sync_copy(accd.at[pl.ds(dbase, 80)],
                        outd_hbm.at[cid, pl.ds(dbase, 80)])

    @pl.when(sid == 15)
    def _():
        pltpu.sync_copy(accd.at[pl.ds(1200, 56)],
                        outd_hbm.at[cid, pl.ds(1200, 56)])


def _attn_sc(q, kv, r, c):
    mesh = plsc.VectorSubcoreMesh(core_axis_name="c", subcore_axis_name="s")
    return pl.kernel(
        _attn_sc_body,
        out_type=[jax.ShapeDtypeStruct((NC, N, EMBED), jnp.float32),
                  jax.ShapeDtypeStruct((NC, ND, EMBED), jnp.float32)],
        mesh=mesh,
        scratch_types=[
            pltpu.VMEM((CA, EMBED), jnp.float32),
            pltpu.VMEM((CA, 2 * EMBED), jnp.float32),
            pltpu.VMEM((CA, EMBED), jnp.float32),
            pltpu.VMEM((2, CA), jnp.int32),
            pltpu.VMEM((2, CA), jnp.int32),
            pltpu.VMEM((2, CA), jnp.int32),
            pltpu.VMEM((2, CA + L), jnp.int32),
            pltpu.SemaphoreType.DMA,
            pltpu.SemaphoreType.DMA,
            pltpu.SemaphoreType.DMA,
            pltpu.VMEM_SHARED((N, EMBED), jnp.float32),
            pltpu.VMEM_SHARED((ND, EMBED), jnp.float32),
        ],
    )(q, kv, r, c)


def _eagg_sc_body(na_hbm, r_hbm, out_hbm, buf, ridx, isem, ssem, acc):
    cid = lax.axis_index("c")
    sid = lax.axis_index("s")
    wid = sid * NC + cid

    zv = jnp.zeros((L,), jnp.float32)

    def zrow(i, _):
        for jj in range(EMBED // L):
            buf[0, i, pl.ds(jj * L, L)] = zv
        return 0
    lax.fori_loop(0, CE, zrow, 0)

    base = sid * ROWS_PER_TILE
    for off, nrow in ((0, 128), (128, 128), (256, 128), (384, 128), (512, 112)):
        pltpu.sync_copy(buf.at[0, pl.ds(0, nrow)], acc.at[pl.ds(base + off, nrow)])

    @pl.when(sid == 0)
    def _():
        pltpu.sync_copy(buf.at[0, pl.ds(0, REM_ROWS)],
                        acc.at[pl.ds(NS * ROWS_PER_TILE, REM_ROWS)])
    plsc.subcore_barrier()

    def issue_load(jn):
        sn = lax.rem(jn, 2)
        ebase = (wid + NW * jn) * CE
        pltpu.async_copy(r_hbm.at[pl.ds(ebase, CE)], ridx.at[sn], isem)
        pltpu.async_copy(na_hbm.at[pl.ds(ebase, CE)], buf.at[sn], isem)

    issue_load(0)

    def chunk(j, _):
        sl = lax.rem(j, 2)
        sp = 1 - sl
        valid = (wid + NW * j) < NCHUNK_E
        valid_n = (wid + NW * (j + 1)) < NCHUNK_E

        @pl.when(j >= 1)
        def _():
            pltpu.make_async_copy(buf.at[sp], acc.at[ridx.at[sp]], ssem).wait()

        @pl.when(valid_n)
        def _():
            issue_load(j + 1)

        @pl.when(valid)
        def _():
            pltpu.make_async_copy(r_hbm.at[pl.ds(0, CE)], ridx.at[sl], isem).wait()
            pltpu.make_async_copy(na_hbm.at[pl.ds(0, CE)], buf.at[sl], isem).wait()
            pltpu.async_copy(buf.at[sl], acc.at[ridx.at[sl]], ssem, add=True)
        return 0

    lax.fori_loop(0, NJ_E, chunk, 0)

    lastsl = (NJ_E - 1) % 2

    @pl.when(wid < NCHUNK_E - NW * (NJ_E - 1))
    def _():
        pltpu.make_async_copy(buf.at[lastsl], acc.at[ridx.at[lastsl]], ssem).wait()

    plsc.subcore_barrier()
    pltpu.sync_copy(acc.at[pl.ds(base, ROWS_PER_TILE)],
                    out_hbm.at[cid, pl.ds(base, ROWS_PER_TILE)])

    @pl.when(sid == 0)
    def _():
        tail = NS * ROWS_PER_TILE
        pltpu.sync_copy(acc.at[pl.ds(tail, REM_ROWS)],
                        out_hbm.at[cid, pl.ds(tail, REM_ROWS)])


def _eagg_sc(node_attr, r):
    mesh = plsc.VectorSubcoreMesh(core_axis_name="c", subcore_axis_name="s")
    return pl.kernel(
        _eagg_sc_body,
        out_type=jax.ShapeDtypeStruct((NC, N, EMBED), jnp.float32),
        mesh=mesh,
        scratch_types=[
            pltpu.VMEM((2, CE, EMBED), jnp.float32),
            pltpu.VMEM((2, CE), jnp.int32),
            pltpu.SemaphoreType.DMA,
            pltpu.SemaphoreType.DMA,
            pltpu.VMEM_SHARED((N, EMBED), jnp.float32),
        ],
    )(node_attr, r)


def _fin_body(feats_ref, acca_ref, accb_ref, eagg_ref, m8_ref,
              wo_ref, bo_ref, w1_ref, b1_ref, w2_ref, b2_ref,
              g_ref, b_ref, o_ref):
    g = g_ref[...]
    b = b_ref[...]
    num = acca_ref[0] + acca_ref[1]
    den = accb_ref[0] + accb_ref[1]
    rec = 1.0 / (den + 1e-16)
    r128 = jax.lax.dot(rec, m8_ref[...], preferred_element_type=jnp.float32)
    agg = num * r128
    att = agg @ wo_ref[...] + bo_ref[...]
    f_ = _ln(_gelu(feats_ref[...] + att), g, b)
    fa = _ln(_gelu(f_ + eagg_ref[0] + eagg_ref[1]), g, b)
    ffn = _gelu(fa @ w1_ref[...] + b1_ref[...]) @ w2_ref[...] + b2_ref[...]
    o_ref[...] = _ln(_gelu(fa + ffn), g, b)


def _finalize(feats, acca, accb, eagg, Wo, bo, W1, b1, W2, b2, ln_g, ln_b):
    blk = 2000
    m8 = jnp.asarray(np.repeat(np.eye(NHEAD, dtype=np.float32), HDIM, axis=1))
    w_spec = lambda shape: pl.BlockSpec(shape, lambda i: (0,) * len(shape))
    return pl.pallas_call(
        _fin_body,
        grid=(N // blk,),
        in_specs=[
            pl.BlockSpec((blk, EMBED), lambda i: (i, 0)),
            pl.BlockSpec((NC, blk, EMBED), lambda i: (0, i, 0)),
            pl.BlockSpec((NC, blk, NHEAD), lambda i: (0, i, 0)),
            pl.BlockSpec((NC, blk, EMBED), lambda i: (0, i, 0)),
            w_spec((NHEAD, EMBED)),
            w_spec((EMBED, EMBED)), w_spec((1, EMBED)),
            w_spec((EMBED, DHID)), w_spec((1, DHID)),
            w_spec((DHID, EMBED)), w_spec((1, EMBED)),
            w_spec((1, EMBED)), w_spec((1, EMBED)),
        ],
        out_specs=pl.BlockSpec((blk, EMBED), lambda i: (i, 0)),
        out_shape=jax.ShapeDtypeStruct((N, EMBED), jnp.float32),
    )(feats, acca, accb, eagg, m8, Wo.T, bo[None], W1.T, b1[None],
      W2.T, b2[None], ln_g[None], ln_b[None])


def kernel(feats, edge_index, edge_attr, Wq, bq, Wk, bk, Wv, bv, Wo, bo,
           We, be, W1, b1, W2, b2, ln_g, ln_b):
    r = edge_index[:, 0]
    c = edge_index[:, 1]
    q, kv = _qkv(feats, Wq, bq, Wk, bk, Wv, bv)
    acca, outd = _attn_sc(q, kv, r, c)
    den = outd.reshape(NC, ND, 8, 16)[:, :, :, :NHEAD].reshape(NC, ND * 8, NHEAD)[:, :N, :]
    na = _node_attr(edge_attr, We, be)
    eagg = _eagg_sc(na, r)
    out = _finalize(feats, acca, den, eagg, Wo, bo, W1, b1, W2, b2,
                    ln_g, ln_b)
    return (out, edge_index, edge_attr)
